# baseline (speedup 1.0000x reference)
import functools

import jax
import jax.numpy as jnp
from jax import lax
from jax.experimental import pallas as pl
from jax.experimental.pallas import tpu as pltpu
from jax.experimental.pallas import tpu_sc as plsc

_VOCAB = 100000
_D = 64
_MAX_NG = 16
_B = 16384
_NC = 2
_NS = 16
_NW = _NC * _NS
_BPW = _B // _NW
_NLANE = 16
_DV = _D // _NLANE
_FCH = 16
_NFCH = _BPW // _FCH
_CH = 16
_NCH = _BPW // _CH


def _sc_body(word_ids_hbm, w_in_hbm, w_ng_hbm, ngm_hbm, ng_len_hbm,
             out_hbm, idx_v, lens_v, ngmi_v, wini_v, ngflat_v, nghalf_v,
             ngm_rows_v, win_rows_v, ng_rows_v, acc_v,
             sem_len, sem_ngm0, sem_ngm1, sem_win0, sem_win1,
             sem_ng0, sem_ng1):
  wid = lax.axis_index("s") * _NC + lax.axis_index("c")
  base = wid * _BPW
  lane = lax.iota(jnp.int32, _NLANE)

  pltpu.sync_copy(word_ids_hbm.at[pl.ds(base, _BPW)], idx_v)

  def idx_body(g, carry):
    wv = idx_v[pl.ds(g * _NLANE, _NLANE)]
    ngmi_v[pl.ds(g * _NLANE, _NLANE)] = wv >> 3
    wini_v[pl.ds(g * _NLANE, _NLANE)] = wv >> 1
    return carry

  lax.fori_loop(0, _BPW // _NLANE, idx_body, 0)
  cp_len = pltpu.async_copy(ng_len_hbm.at[idx_v], lens_v, sem_len)

  ngm_sems = [sem_ngm0, sem_ngm1]

  def start_ngm(c, buf):
    return pltpu.async_copy(
        ngm_hbm.at[ngmi_v.at[pl.ds(c * _FCH, _FCH)]],
        ngm_rows_v.at[buf], ngm_sems[buf])

  start_ngm(0, 0)

  def flat_chunk(c, carry):
    for b in range(2):
      cc = c * 2 + b
      pltpu.make_async_copy(
          ngm_hbm.at[ngmi_v.at[pl.ds(cc * _FCH, _FCH)]],
          ngm_rows_v.at[b], ngm_sems[b]).wait()

      @pl.when(cc + 1 < _NFCH)
      def _(cc=cc, b=b):
        start_ngm(cc + 1, 1 - b)

      for g in range(_FCH // _NLANE):
        wv = idx_v[pl.ds(cc * _FCH + g * _NLANE, _NLANE)]
        for wi in range(_NLANE):
          ws = g * _NLANE + wi
          sub = wv[wi] & 7
          idv = ngm_rows_v[b, ws, pl.ds(sub * _NLANE, _NLANE)]
          ngflat_v[pl.ds((cc * _FCH + ws) * _MAX_NG, _MAX_NG)] = idv >> 1
          nghalf_v[pl.ds((cc * _FCH + ws) * _MAX_NG, _MAX_NG)] = idv & 1
    return carry

  lax.fori_loop(0, _NFCH // 2, flat_chunk, 0)
  cp_len.wait()

  rtab = jnp.full((_NLANE,), 1.0 / (1.0 + _MAX_NG), dtype=jnp.float32)
  for k in range(_MAX_NG - 1):
    rtab = jnp.where(lane == k, jnp.float32(1.0 / (2.0 + k)), rtab)

  win_sems = [sem_win0, sem_win1]
  ng_sems = [sem_ng0, sem_ng1]

  def start_rows(c, buf):
    pltpu.async_copy(
        w_in_hbm.at[wini_v.at[pl.ds(c * _CH, _CH)]],
        win_rows_v.at[buf], win_sems[buf])
    pltpu.async_copy(
        w_ng_hbm.at[ngflat_v.at[pl.ds(c * _CH * _MAX_NG, _CH * _MAX_NG)]],
        ng_rows_v.at[buf], ng_sems[buf])

  start_rows(0, 0)
  start_rows(1, 1)

  def chunk(c2, carry):
    for b in range(2):
      c = c2 * 2 + b
      pltpu.make_async_copy(
          w_in_hbm.at[wini_v.at[pl.ds(c * _CH, _CH)]],
          win_rows_v.at[b], win_sems[b]).wait()
      pltpu.make_async_copy(
          w_ng_hbm.at[ngflat_v.at[pl.ds(c * _CH * _MAX_NG, _CH * _MAX_NG)]],
          ng_rows_v.at[b], ng_sems[b]).wait()

      wv = idx_v[pl.ds(c * _NLANE, _NLANE)]
      lv = lens_v[pl.ds(c * _NLANE, _NLANE)]
      invs = jnp.take(rtab, jnp.clip(lv - 1, 0, _MAX_NG - 1), mode="fill")
      for wi in range(_NLANE):
        hin = (wv[wi] & 1) * _D
        lnc = jnp.minimum(lv[wi], _MAX_NG)
        w = c * _CH + wi
        offv = nghalf_v[pl.ds(w * _MAX_NG, _MAX_NG)] * _D
        accs = tuple(
            win_rows_v[b, wi, pl.ds(hin + d * _NLANE, _NLANE)]
            for d in range(_DV))

        def j_body(j, accs, b=b, wi=wi, offv=offv):
          jv = jnp.full((_NLANE,), j, dtype=jnp.int32)
          off = jnp.take(offv, jv, mode="fill")
          rows = jv + wi * _MAX_NG
          return tuple(
              accs[d] + plsc.load_gather(
                  ng_rows_v.at[b], [rows, off + (d * _NLANE + lane)])
              for d in range(_DV))

        accs = lax.fori_loop(0, lnc, j_body, accs)
        inv = jnp.take(invs, jnp.full((_NLANE,), wi, dtype=jnp.int32),
                       mode="fill")
        arow = c * (_CH // 2) + wi // 2
        aoff = (wi & 1) * _D
        for d in range(_DV):
          acc_v[arow, pl.ds(aoff + d * _NLANE, _NLANE)] = accs[d] * inv

      @pl.when(c + 2 < _NCH)
      def _(c=c, b=b):
        start_rows(c + 2, b)
    return carry

  lax.fori_loop(0, _NCH // 2, chunk, 0)
  pltpu.sync_copy(acc_v, out_hbm.at[pl.ds(wid * (_BPW // 2), _BPW // 2)])


@jax.jit
def kernel(word_ids, W_in, W_ng, ng_matrix, ng_lengths):
  mesh = plsc.VectorSubcoreMesh(core_axis_name="c", subcore_axis_name="s")
  run = functools.partial(
      pl.kernel,
      out_type=jax.ShapeDtypeStruct((_B // 2, 2 * _D), jnp.float32),
      mesh=mesh,
      compiler_params=pltpu.CompilerParams(needs_layout_passes=False),
      scratch_types=[
          pltpu.VMEM((_BPW,), jnp.int32),
          pltpu.VMEM((_BPW,), jnp.int32),
          pltpu.VMEM((_BPW,), jnp.int32),
          pltpu.VMEM((_BPW,), jnp.int32),
          pltpu.VMEM((_BPW * _MAX_NG,), jnp.int32),
          pltpu.VMEM((_BPW * _MAX_NG,), jnp.int32),
          pltpu.VMEM((2, _FCH, 2 * _D), jnp.int32),
          pltpu.VMEM((2, _CH, 2 * _D), jnp.float32),
          pltpu.VMEM((2, _CH * _MAX_NG, 2 * _D), jnp.float32),
          pltpu.VMEM((_BPW // 2, 2 * _D), jnp.float32),
          pltpu.SemaphoreType.DMA,
          pltpu.SemaphoreType.DMA,
          pltpu.SemaphoreType.DMA,
          pltpu.SemaphoreType.DMA,
          pltpu.SemaphoreType.DMA,
          pltpu.SemaphoreType.DMA,
          pltpu.SemaphoreType.DMA,
      ],
  )(_sc_body)
  out2 = run(word_ids, W_in.reshape(_VOCAB // 2, 2 * _D),
             W_ng.reshape(250000, 2 * _D),
             ng_matrix.reshape(_VOCAB // 8, 8 * _MAX_NG),
             ng_lengths)
  return out2.reshape(_B, _D)
